# baseline (speedup 1.0000x reference)
import functools

import jax
import jax.numpy as jnp
from jax import lax
from jax.experimental import pallas as pl
from jax.experimental.pallas import tpu as pltpu
from jax.experimental.pallas import tpu_sc as plsc

NC, NS, L = 2, 16, 16
NW = NC * NS
MBLK = 512
N, D, E, TOPK = 8192, 1024, 8, 2
A = N * TOPK
PT = A + E * MBLK
NB = PT // MBLK
NBP = 48
DW = D // 2


def _router_kernel(x_ref, wg_ref, bg_row_ref, bg_col_ref,
                   logits_ref, w_ref, idx_ref, mask_ref, rank_ref, counts_ref,
                   carry, tri):
    i = pl.program_id(0)
    blk = x_ref.shape[0]

    @pl.when(i == 0)
    def _init():
        r = lax.broadcasted_iota(jnp.int32, (blk, blk), 0)
        c = lax.broadcasted_iota(jnp.int32, (blk, blk), 1)
        tri[...] = (c < r).astype(jnp.float32)
        carry[...] = jnp.zeros((1, E), jnp.float32)

    x = x_ref[...]
    wg = wg_ref[...]
    logits = lax.dot_general(
        x, wg, (((1,), (1,)), ((), ())),
        preferred_element_type=jnp.float32) + bg_row_ref[...]
    logits_ref[...] = logits
    mx = jnp.max(logits, axis=1, keepdims=True)
    ex = jnp.exp(logits - mx)
    probs = ex / jnp.sum(ex, axis=1, keepdims=True)
    iota_e = lax.broadcasted_iota(jnp.int32, probs.shape, 1)
    p0 = jnp.max(probs, axis=1, keepdims=True)
    i0 = jnp.min(jnp.where(probs == p0, iota_e, E), axis=1, keepdims=True)
    probs1 = jnp.where(iota_e == i0, -1.0, probs)
    p1 = jnp.max(probs1, axis=1, keepdims=True)
    i1 = jnp.min(jnp.where(probs1 == p1, iota_e, E), axis=1, keepdims=True)
    s = p0 + p1
    w0 = p0 / s
    w1 = p1 / s
    w_ref[:, 0:1] = w0
    w_ref[:, 1:2] = w1
    idx_ref[:, 0:1] = i0
    idx_ref[:, 1:2] = i1

    oh0 = (iota_e == i0).astype(jnp.float32)
    oh1 = (iota_e == i1).astype(jnp.float32)
    ohs = oh0 + oh1
    pref = lax.dot_general(tri[...], ohs, (((1,), (0,)), ((), ())),
                           preferred_element_type=jnp.float32)
    base = carry[...] + pref
    rank_ref[:, 0:1] = jnp.sum(oh0 * base, axis=1, keepdims=True).astype(jnp.int32)
    rank_ref[:, 1:2] = jnp.sum(oh1 * base, axis=1, keepdims=True).astype(jnp.int32)
    carry[...] = carry[...] + jnp.sum(ohs, axis=0, keepdims=True)
    counts_ref[...] = carry[...].astype(jnp.int32)

    logits_t = lax.dot_general(
        wg, x, (((1,), (1,)), ((), ())),
        preferred_element_type=jnp.float32) + bg_col_ref[...]
    mx_t = jnp.max(logits_t, axis=0, keepdims=True)
    ex_t = jnp.exp(logits_t - mx_t)
    probs_t = ex_t / jnp.sum(ex_t, axis=0, keepdims=True)
    iota_t = lax.broadcasted_iota(jnp.int32, probs_t.shape, 0)
    p0_t = jnp.max(probs_t, axis=0, keepdims=True)
    i0_t = jnp.min(jnp.where(probs_t == p0_t, iota_t, E), axis=0, keepdims=True)
    probs1_t = jnp.where(iota_t == i0_t, -1.0, probs_t)
    p1_t = jnp.max(probs1_t, axis=0, keepdims=True)
    i1_t = jnp.min(jnp.where(probs1_t == p1_t, iota_t, E), axis=0, keepdims=True)
    mask_ref[:, 0, :] = (iota_t == i0_t).astype(jnp.int32)
    mask_ref[:, 1, :] = (iota_t == i1_t).astype(jnp.int32)


def _run_router(h, Wg, bg):
    blk = 1024
    grid = (N // blk,)
    out_shapes = (
        jax.ShapeDtypeStruct((N, E), jnp.float32),
        jax.ShapeDtypeStruct((N, 2), jnp.float32),
        jax.ShapeDtypeStruct((N, 2), jnp.int32),
        jax.ShapeDtypeStruct((E, 2, N), jnp.int32),
        jax.ShapeDtypeStruct((N, 2), jnp.int32),
        jax.ShapeDtypeStruct((1, E), jnp.int32),
    )
    return pl.pallas_call(
        _router_kernel,
        grid=grid,
        in_specs=[
            pl.BlockSpec((blk, D), lambda i: (i, 0)),
            pl.BlockSpec((E, D), lambda i: (0, 0)),
            pl.BlockSpec((1, E), lambda i: (0, 0)),
            pl.BlockSpec((E, 1), lambda i: (0, 0)),
        ],
        out_specs=(
            pl.BlockSpec((blk, E), lambda i: (i, 0)),
            pl.BlockSpec((blk, 2), lambda i: (i, 0)),
            pl.BlockSpec((blk, 2), lambda i: (i, 0)),
            pl.BlockSpec((E, 2, blk), lambda i: (0, 0, i)),
            pl.BlockSpec((blk, 2), lambda i: (i, 0)),
            pl.BlockSpec((1, E), lambda i: (0, 0)),
        ),
        out_shape=out_shapes,
        scratch_shapes=[
            pltpu.VMEM((1, E), jnp.float32),
            pltpu.VMEM((blk, blk), jnp.float32),
        ],
    )(h, Wg, bg.reshape(1, E), bg.reshape(E, 1))


_SC_MESH = plsc.VectorSubcoreMesh(
    core_axis_name="c", subcore_axis_name="s", num_cores=NC, num_subcores=NS)

_AW = A // NW
_AR = _AW // 128


@functools.partial(
    pl.kernel,
    out_type=(
        jax.ShapeDtypeStruct((A // 128, 128), jnp.int32),
        jax.ShapeDtypeStruct((PT,), jnp.int32),
        jax.ShapeDtypeStruct((PT,), jnp.float32),
        jax.ShapeDtypeStruct((NBP,), jnp.int32),
    ),
    mesh=_SC_MESH,
    scratch_types=[
        pltpu.VMEM((_AR, 128), jnp.int32),
        pltpu.VMEM((_AR, 128), jnp.int32),
        pltpu.VMEM((_AR, 128), jnp.float32),
        pltpu.VMEM((_AR, 128), jnp.int32),
        pltpu.VMEM((_AR, 128), jnp.int32),
        pltpu.VMEM((16,), jnp.int32),
        pltpu.VMEM((16,), jnp.int32),
        pltpu.VMEM((16,), jnp.int32),
        pltpu.VMEM((NBP,), jnp.int32),
        pltpu.VMEM((NBP,), jnp.int32),
    ],
    compiler_params=pltpu.CompilerParams(needs_layout_passes=False),
)
def _dispatch_sc(idx_hbm, rank_hbm, w_hbm, counts_hbm,
                 pos_hbm, tsrc_hbm, wsrc_hbm, bexp_hbm,
                 ev, rv, wv, pv, tv, offv, cumv, cv, bev, hv):
    wid = lax.axis_index("s") * NC + lax.axis_index("c")
    row0 = wid * _AR

    pltpu.sync_copy(idx_hbm.at[pl.ds(row0, _AR), :], ev)
    pltpu.sync_copy(rank_hbm.at[pl.ds(row0, _AR), :], rv)
    pltpu.sync_copy(w_hbm.at[pl.ds(row0, _AR), :], wv)
    pltpu.sync_copy(counts_hbm, cv.at[pl.ds(0, 8)])

    lane = lax.iota(jnp.int32, 16)
    c = jnp.where(lane < 8, cv[...], 0)
    padded = ((c + (MBLK - 1)) >> 9) << 9
    cum = plsc.cumsum(padded)
    offv[...] = cum - padded
    cumv[...] = cum

    for j in range(_AR):
        for k in range(8):
            col = k * 16
            e16 = ev[j, pl.ds(col, 16)]
            r16 = rv[j, pl.ds(col, 16)]
            off16 = plsc.load_gather(offv, [e16])
            pv[j, pl.ds(col, 16)] = off16 + r16
            aidx = (row0 + j) * 128 + col + lane
            tv[j, pl.ds(col, 16)] = aidx >> 1

    pltpu.sync_copy(pv, pos_hbm.at[pl.ds(row0, _AR), :])
    for j in range(_AR):
        pltpu.sync_copy(tv.at[j], tsrc_hbm.at[pv.at[j]])
        pltpu.sync_copy(wv.at[j], wsrc_hbm.at[pv.at[j]])

    @pl.when(wid == 0)
    def _block_expert():
        zero16 = jnp.zeros((16,), jnp.int32)
        one16 = zero16 + 1
        for j in range(NBP // 16):
            hv[pl.ds(j * 16, 16)] = zero16
        bidx = lax.shift_right_logical(cum, 9)
        for e in range(E):
            plsc.addupdate_scatter(hv, [bidx], one16, mask=lane == e)
        carry = 0
        for j in range(NBP // 16):
            chunk = hv[pl.ds(j * 16, 16)]
            bev[pl.ds(j * 16, 16)] = plsc.cumsum(chunk) + carry
            carry = carry + jnp.sum(chunk)
        pltpu.sync_copy(bev, bexp_hbm)


_RW = PT // NW
_GC = 64
_NGC = _RW // _GC


@functools.partial(
    pl.kernel,
    out_type=jax.ShapeDtypeStruct((PT, DW), jnp.int32),
    mesh=_SC_MESH,
    scratch_types=[
        pltpu.VMEM((_RW,), jnp.int32),
        pltpu.VMEM((_GC, DW), jnp.int32),
        pltpu.SemaphoreType.DMA,
    ],
    compiler_params=pltpu.CompilerParams(needs_layout_passes=False),
)
def _gather_sc(x_hbm, tsrc_hbm, xp_hbm, idxc, rows, sem):
    wid = lax.axis_index("s") * NC + lax.axis_index("c")
    base = wid * _RW
    pltpu.sync_copy(tsrc_hbm.at[pl.ds(base, _RW)], idxc)
    for o in range(0, _RW, 16):
        v = idxc[pl.ds(o, 16)]
        idxc[pl.ds(o, 16)] = jnp.clip(v, 0, N - 1)
    for ci in range(_NGC):
        pltpu.async_copy(
            x_hbm.at[idxc.at[pl.ds(ci * _GC, _GC)]], rows, sem).wait()
        pltpu.sync_copy(rows, xp_hbm.at[pl.ds(base + ci * _GC, _GC), :])


def _gmm_kernel(bexp_ref, xp_ref, we_ref, be_ref, w_ref, y_ref):
    i = pl.program_id(0)

    @pl.when(bexp_ref[i] < E)
    def _do():
        y = lax.dot_general(
            xp_ref[...], we_ref[0], (((1,), (1,)), ((), ())),
            preferred_element_type=jnp.float32) + be_ref[0]
        y_ref[...] = (y * w_ref[...]).astype(jnp.bfloat16)


def _run_gmm(bexp, xp_bf, We_bf, be, wsrc):
    def we_map(i, b):
        return (jnp.minimum(b[i], E - 1), 0, 0)

    grid_spec = pltpu.PrefetchScalarGridSpec(
        num_scalar_prefetch=1,
        grid=(NB,),
        in_specs=[
            pl.BlockSpec((MBLK, D), lambda i, b: (i, 0)),
            pl.BlockSpec((1, D, D), we_map),
            pl.BlockSpec((1, 1, D), we_map),
            pl.BlockSpec((MBLK, 1), lambda i, b: (i, 0)),
        ],
        out_specs=pl.BlockSpec((MBLK, D), lambda i, b: (i, 0)),
    )
    return pl.pallas_call(
        _gmm_kernel,
        grid_spec=grid_spec,
        out_shape=jax.ShapeDtypeStruct((PT, D), jnp.bfloat16),
    )(bexp, xp_bf, We_bf, be.reshape(E, 1, D), wsrc.reshape(PT, 1))


_TW = N // NW
_CC = 32
_NCC = _TW // _CC


@functools.partial(
    pl.kernel,
    out_type=jax.ShapeDtypeStruct((N, DW), jnp.int32),
    mesh=_SC_MESH,
    scratch_types=[
        pltpu.VMEM((_AR, 128), jnp.int32),
        pltpu.VMEM((2 * _CC, DW), jnp.int32),
        pltpu.VMEM((_CC, DW), jnp.int32),
        pltpu.SemaphoreType.DMA,
    ],
    compiler_params=pltpu.CompilerParams(needs_layout_passes=False),
)
def _combine_sc(y_hbm, pos_hbm, fin_hbm, posc, rows, outc, sem):
    wid = lax.axis_index("s") * NC + lax.axis_index("c")
    tok0 = wid * _TW
    pltpu.sync_copy(pos_hbm.at[pl.ds(wid * _AR, _AR), :], posc)
    for ci in range(_NCC):
        pltpu.async_copy(
            y_hbm.at[posc.at[ci // 2, pl.ds((ci % 2) * 64, 2 * _CC)]],
            rows, sem).wait()

        def body(t, _):
            for s in range(DW // 16):
                a = rows[2 * t, pl.ds(s * 16, 16)]
                b = rows[2 * t + 1, pl.ds(s * 16, 16)]
                r = plsc.bitcast(
                    plsc.bitcast(a, jnp.bfloat16) + plsc.bitcast(b, jnp.bfloat16),
                    jnp.int32)
                outc[t, pl.ds(s * 16, 16)] = r
            return _

        lax.fori_loop(0, _CC, body, None)
        pltpu.sync_copy(outc, fin_hbm.at[pl.ds(tok0 + ci * _CC, _CC), :])


def kernel(x, Wg, bg, We, be):
    b, s, d = x.shape
    h = x.reshape(N, D)

    logits, weights, indices, mask, rank, counts = _run_router(h, Wg, bg)

    idx2d = indices.reshape(A // 128, 128)
    rank2d = rank.reshape(A // 128, 128)
    w2d = weights.reshape(A // 128, 128)
    pos2d, tsrc, wsrc, bexp = _dispatch_sc(
        idx2d, rank2d, w2d, counts.reshape(E))

    x_bf = h.astype(jnp.bfloat16)
    xi32 = lax.bitcast_convert_type(x_bf.reshape(N, DW, 2), jnp.int32)
    xp_i32 = _gather_sc(xi32, tsrc)
    xp_bf = lax.bitcast_convert_type(
        xp_i32.reshape(PT, DW, 1), jnp.bfloat16).reshape(PT, D)

    y_bf = _run_gmm(bexp, xp_bf, We.astype(jnp.bfloat16), be, wsrc)
    yi32 = lax.bitcast_convert_type(y_bf.reshape(PT, DW, 2), jnp.int32)

    fin_i32 = _combine_sc(yi32, pos2d)
    final = lax.bitcast_convert_type(
        fin_i32.reshape(N, DW, 1), jnp.bfloat16).reshape(N, D)
    final = final.astype(jnp.float32).reshape(b, s, d)

    return (final, logits, weights, indices, mask)


# speedup vs baseline: 7.1929x; 7.1929x over previous
import functools

import jax
import jax.numpy as jnp
from jax.experimental import pallas as pl
from jax.experimental.pallas import tpu as pltpu


def _moe_kernel(x_ref, wg_ref, bg_row_ref, bg_col_ref, we_ref, be_ref,
                out_ref, logits_ref, w_ref, idx_ref, mask_ref,
                wscr, iscr, xbf):
    e = pl.program_id(1)
    num_e = pl.num_programs(1)

    @pl.when(e == 0)
    def _router():
        x = x_ref[...]
        xbf[...] = x.astype(jnp.bfloat16)
        wg = wg_ref[...]
        logits = jax.lax.dot_general(
            x, wg, (((1,), (1,)), ((), ())),
            preferred_element_type=jnp.float32) + bg_row_ref[...]
        logits_ref[...] = logits
        mx = jnp.max(logits, axis=1, keepdims=True)
        ex = jnp.exp(logits - mx)
        probs = ex / jnp.sum(ex, axis=1, keepdims=True)
        iota_e = jax.lax.broadcasted_iota(jnp.int32, probs.shape, 1)
        p0 = jnp.max(probs, axis=1, keepdims=True)
        i0 = jnp.min(jnp.where(probs == p0, iota_e, num_e),
                     axis=1, keepdims=True)
        probs1 = jnp.where(iota_e == i0, -1.0, probs)
        p1 = jnp.max(probs1, axis=1, keepdims=True)
        i1 = jnp.min(jnp.where(probs1 == p1, iota_e, num_e),
                     axis=1, keepdims=True)
        s = p0 + p1
        w0 = p0 / s
        w1 = p1 / s
        w_ref[:, 0:1] = w0
        w_ref[:, 1:2] = w1
        idx_ref[:, 0:1] = i0
        idx_ref[:, 1:2] = i1
        wscr[:, 0:1] = w0
        wscr[:, 1:2] = w1
        iscr[:, 0:1] = i0
        iscr[:, 1:2] = i1
        logits_t = jax.lax.dot_general(
            wg, x, (((1,), (1,)), ((), ())),
            preferred_element_type=jnp.float32) + bg_col_ref[...]
        mx_t = jnp.max(logits_t, axis=0, keepdims=True)
        ex_t = jnp.exp(logits_t - mx_t)
        probs_t = ex_t / jnp.sum(ex_t, axis=0, keepdims=True)
        iota_t = jax.lax.broadcasted_iota(jnp.int32, probs_t.shape, 0)
        p0_t = jnp.max(probs_t, axis=0, keepdims=True)
        i0_t = jnp.min(jnp.where(probs_t == p0_t, iota_t, num_e),
                       axis=0, keepdims=True)
        probs1_t = jnp.where(iota_t == i0_t, -1.0, probs_t)
        p1_t = jnp.max(probs1_t, axis=0, keepdims=True)
        i1_t = jnp.min(jnp.where(probs1_t == p1_t, iota_t, num_e),
                       axis=0, keepdims=True)
        mask_ref[:, 0, :] = (iota_t == i0_t).astype(jnp.int32)
        mask_ref[:, 1, :] = (iota_t == i1_t).astype(jnp.int32)

    w0 = wscr[:, 0:1]
    w1 = wscr[:, 1:2]
    i0 = iscr[:, 0:1]
    i1 = iscr[:, 1:2]
    we = we_ref[0]
    eo = jax.lax.dot_general(
        xbf[...], we, (((1,), (1,)), ((), ())),
        preferred_element_type=jnp.float32) + be_ref[0]
    w_e = jnp.where(i0 == e, w0, 0.0) + jnp.where(i1 == e, w1, 0.0)
    contrib = eo * w_e

    @pl.when(e == 0)
    def _init():
        out_ref[...] = contrib

    @pl.when(e > 0)
    def _acc():
        out_ref[...] = out_ref[...] + contrib


def kernel(x, Wg, bg, We, be):
    b, s, d = x.shape
    n = b * s
    num_e = Wg.shape[0]
    h = x.reshape(n, d)
    blk = 2048 if n % 2048 == 0 else n
    grid = (n // blk, num_e)

    out_shapes = (
        jax.ShapeDtypeStruct((n, d), jnp.float32),
        jax.ShapeDtypeStruct((n, num_e), jnp.float32),
        jax.ShapeDtypeStruct((n, 2), jnp.float32),
        jax.ShapeDtypeStruct((n, 2), jnp.int32),
        jax.ShapeDtypeStruct((num_e, 2, n), jnp.int32),
    )
    final, logits, weights, indices, mask = pl.pallas_call(
        _moe_kernel,
        grid=grid,
        in_specs=[
            pl.BlockSpec((blk, d), lambda i, e: (i, 0)),
            pl.BlockSpec((num_e, d), lambda i, e: (0, 0)),
            pl.BlockSpec((1, num_e), lambda i, e: (0, 0)),
            pl.BlockSpec((num_e, 1), lambda i, e: (0, 0)),
            pl.BlockSpec((1, d, d), lambda i, e: (e, 0, 0)),
            pl.BlockSpec((1, 1, d), lambda i, e: (e, 0, 0)),
        ],
        out_specs=(
            pl.BlockSpec((blk, d), lambda i, e: (i, 0)),
            pl.BlockSpec((blk, num_e), lambda i, e: (i, 0)),
            pl.BlockSpec((blk, 2), lambda i, e: (i, 0)),
            pl.BlockSpec((blk, 2), lambda i, e: (i, 0)),
            pl.BlockSpec((num_e, 2, blk), lambda i, e: (0, 0, i)),
        ),
        out_shape=out_shapes,
        scratch_shapes=[
            pltpu.VMEM((blk, 2), jnp.float32),
            pltpu.VMEM((blk, 2), jnp.int32),
            pltpu.VMEM((blk, d), jnp.bfloat16),
        ],
        compiler_params=pltpu.CompilerParams(
            vmem_limit_bytes=100 * 1024 * 1024),
    )(h, Wg, bg.reshape(1, num_e), bg.reshape(num_e, 1),
      We.astype(jnp.bfloat16), be.reshape(num_e, 1, d))

    return (final.reshape(b, s, d), logits, weights, indices, mask)
